# Initial kernel scaffold; baseline (speedup 1.0000x reference)
import jax
import jax.numpy as jnp
from jax.experimental import pallas as pl


def kernel(hist, logs):
    raise NotImplementedError("write your pallas kernel here")


# speedup vs baseline: 5.3185x; 5.3185x over previous
import jax
import jax.numpy as jnp
from jax.experimental import pallas as pl


def _bcast_body(logs_ref, out_ref):
    out_ref[...] = jnp.broadcast_to(logs_ref[...], out_ref.shape)


def kernel(hist, logs):
    S_, B_ = hist.shape
    V = logs.shape[0]
    rows = (S_ + 1) * B_
    grid = 16
    block_rows = rows // grid
    assert block_rows * grid == rows

    logs2d = logs.reshape(1, V)
    out = pl.pallas_call(
        _bcast_body,
        grid=(grid,),
        in_specs=[pl.BlockSpec((1, V), lambda i: (0, 0))],
        out_specs=pl.BlockSpec((block_rows, V), lambda i: (i, 0)),
        out_shape=jax.ShapeDtypeStruct((rows, V), logs.dtype),
    )(logs2d)
    return out.reshape(S_ + 1, B_, V)


# speedup vs baseline: 5.3692x; 1.0095x over previous
import jax
import jax.numpy as jnp
from jax.experimental import pallas as pl
from jax.experimental.pallas import tpu as pltpu

_CHUNKS = 16


def _bcast_body(logs_ref, out_ref, buf_ref, sems):
    buf_ref[...] = jnp.broadcast_to(logs_ref[...], buf_ref.shape)
    rows = buf_ref.shape[0]
    for i in range(_CHUNKS):
        pltpu.make_async_copy(
            buf_ref, out_ref.at[pl.ds(i * rows, rows), :], sems.at[i]
        ).start()
    for i in range(_CHUNKS):
        pltpu.make_async_copy(
            buf_ref, out_ref.at[pl.ds(i * rows, rows), :], sems.at[i]
        ).wait()


def kernel(hist, logs):
    S_, B_ = hist.shape
    V = logs.shape[0]
    rows = (S_ + 1) * B_
    chunk_rows = rows // _CHUNKS
    assert chunk_rows * _CHUNKS == rows

    logs2d = logs.reshape(1, V)
    out = pl.pallas_call(
        _bcast_body,
        in_specs=[pl.BlockSpec(memory_space=pltpu.VMEM)],
        out_specs=pl.BlockSpec(memory_space=pl.ANY),
        out_shape=jax.ShapeDtypeStruct((rows, V), logs.dtype),
        scratch_shapes=[
            pltpu.VMEM((chunk_rows, V), logs.dtype),
            pltpu.SemaphoreType.DMA((_CHUNKS,)),
        ],
    )(logs2d)
    return out.reshape(S_ + 1, B_, V)
